# baseline (speedup 1.0000x reference)
import functools

import jax
import jax.numpy as jnp
from jax import lax
from jax.experimental import pallas as pl
from jax.experimental.pallas import tpu as pltpu
from jax.experimental.pallas import tpu_sc as plsc

N_NODES = 10000
N_EDGES = 320000
D_IN = 128
HIDDEN = 512

NC = 2
NS = 16
NW = NC * NS
CHUNK = 128
NCHUNK = 80
EPW = CHUNK * NCHUNK
E_PAD = NW * EPW
N_ACC = 10080

ZSTRIDE = 624
ZSIZE = 640


def _sc_agg(x, src3, dst3, zeros_agg):
    mesh = plsc.VectorSubcoreMesh(core_axis_name="c", subcore_axis_name="s")

    @functools.partial(
        pl.kernel,
        mesh=mesh,
        out_type=(
            jax.ShapeDtypeStruct((NC, N_NODES, D_IN), jnp.float32),
            jax.ShapeDtypeStruct((NC, N_NODES, D_IN), jnp.float32),
        ),
        scratch_types=[
            pltpu.VMEM((CHUNK,), jnp.int32),
            pltpu.VMEM((CHUNK,), jnp.int32),
            pltpu.VMEM((CHUNK,), jnp.int32),
            pltpu.VMEM((CHUNK,), jnp.int32),
            pltpu.VMEM((CHUNK,), jnp.int32),
            pltpu.VMEM((CHUNK,), jnp.int32),
            pltpu.VMEM((CHUNK, D_IN), jnp.float32),
            pltpu.VMEM((CHUNK, D_IN), jnp.float32),
            pltpu.VMEM((CHUNK, D_IN), jnp.float32),
            pltpu.VMEM_SHARED((N_ACC, D_IN), jnp.float32),
            pltpu.SemaphoreType.DMA,
            pltpu.SemaphoreType.DMA,
            pltpu.SemaphoreType.DMA,
            pltpu.SemaphoreType.DMA,
            pltpu.SemaphoreType.DMA,
            pltpu.SemaphoreType.DMA,
        ],
    )
    def k(x_hbm, src_hbm, dst_hbm, zagg_hbm, agg_out, deg_out,
          src_a, src_b, src_c, dst_a, dst_b, dst_c,
          rows_a, rows_b, rows_c, agg_sh,
          gsa, gsb, gsc, ssa, ssb, ssc):
        c = lax.axis_index("c")
        s = lax.axis_index("s")
        wid = c * NS + s
        row0 = s * ZSTRIDE

        pltpu.sync_copy(zagg_hbm.at[pl.ds(row0, ZSIZE)],
                        agg_sh.at[pl.ds(row0, ZSIZE)])
        plsc.subcore_barrier()

        srcv = (src_a, src_b, src_c)
        dstv = (dst_a, dst_b, dst_c)
        rows = (rows_a, rows_b, rows_c)
        gs = (gsa, gsb, gsc)
        ss = (ssa, ssb, ssc)

        def load_idx(g, b):
            pltpu.sync_copy(src_hbm.at[wid, g], srcv[b])
            pltpu.sync_copy(dst_hbm.at[wid, g], dstv[b])

        def start_gather(b):
            pltpu.async_copy(x_hbm.at[srcv[b]], rows[b], gs[b])

        def wait_gather(b):
            pltpu.make_async_copy(x_hbm.at[srcv[b]], rows[b], gs[b]).wait()

        def start_scatter(b):
            pltpu.async_copy(rows[b], agg_sh.at[dstv[b]], ss[b], add=True)

        def wait_scatter(b):
            pltpu.make_async_copy(rows[b], agg_sh.at[dstv[b]], ss[b]).wait()

        load_idx(0, 0)
        start_gather(0)
        load_idx(1, 1)
        start_gather(1)
        wait_gather(0)
        start_scatter(0)

        def stage(g, bf, bg, bs):
            load_idx(g, bf)
            start_gather(bf)
            wait_gather(bg)
            start_scatter(bg)
            wait_scatter(bs)

        def body(g, carry):
            @pl.when(lax.rem(g, 3) == 2)
            def _():
                stage(g, 2, 1, 0)
            @pl.when(lax.rem(g, 3) == 0)
            def _():
                stage(g, 0, 2, 1)
            @pl.when(lax.rem(g, 3) == 1)
            def _():
                stage(g, 1, 0, 2)
            return carry
        lax.fori_loop(2, NCHUNK, body, 0)

        wait_gather(1)
        start_scatter(1)
        wait_scatter(0)
        wait_scatter(1)

        plsc.subcore_barrier()
        pltpu.sync_copy(agg_sh.at[pl.ds(row0, ZSIZE)],
                        agg_out.at[c, pl.ds(row0, ZSIZE)])
        plsc.subcore_barrier()

        pltpu.sync_copy(zagg_hbm.at[pl.ds(row0, ZSIZE)],
                        agg_sh.at[pl.ds(row0, ZSIZE)])

        one16 = jnp.ones((16,), jnp.float32)

        def fill_ones(i, carry):
            def fill_col(j, carry2):
                rows_a[i, pl.ds(j * 16, 16)] = one16
                return carry2
            return lax.fori_loop(0, D_IN // 16, fill_col, carry)
        lax.fori_loop(0, CHUNK, fill_ones, 0)

        plsc.subcore_barrier()

        def load_dst(g, b):
            pltpu.sync_copy(dst_hbm.at[wid, g], dstv[b])

        def start_dscatter(b):
            pltpu.async_copy(rows_a, agg_sh.at[dstv[b]], ss[b], add=True)

        def wait_dscatter(b):
            pltpu.make_async_copy(rows_a, agg_sh.at[dstv[b]], ss[b]).wait()

        load_dst(0, 0)
        start_dscatter(0)
        load_dst(1, 1)
        start_dscatter(1)
        load_dst(2, 2)
        start_dscatter(2)

        def dstage(g, b):
            wait_dscatter(b)
            load_dst(g, b)
            start_dscatter(b)

        def dbody(g, carry):
            @pl.when(lax.rem(g, 3) == 0)
            def _():
                dstage(g, 0)
            @pl.when(lax.rem(g, 3) == 1)
            def _():
                dstage(g, 1)
            @pl.when(lax.rem(g, 3) == 2)
            def _():
                dstage(g, 2)
            return carry
        lax.fori_loop(3, NCHUNK, dbody, 0)

        wait_dscatter(0)
        wait_dscatter(1)
        wait_dscatter(2)

        plsc.subcore_barrier()
        pltpu.sync_copy(agg_sh.at[pl.ds(row0, ZSIZE)],
                        deg_out.at[c, pl.ds(row0, ZSIZE)])

    return k(x, src3, dst3, zeros_agg)


def _tc_dense(x, agg2, deg2, W_self, W_neigh, b_conv, W_proj, b_proj):
    R = 1000
    grid = (N_NODES // R,)

    def body(x_ref, agg_ref, deg_ref, ws_ref, wn_ref, bc_ref, wp_ref, bp_ref,
             out_ref):
        agg = agg_ref[0] + agg_ref[1]
        deg = (deg_ref[0] + deg_ref[1])[:, 0:1]
        mean = agg / jnp.maximum(deg, 1.0)
        h = jnp.dot(x_ref[...], ws_ref[...], preferred_element_type=jnp.float32)
        h = h + jnp.dot(mean, wn_ref[...], preferred_element_type=jnp.float32)
        h = jnp.maximum(h + bc_ref[...], 0.0)
        out_ref[...] = (jnp.dot(h, wp_ref[...],
                                preferred_element_type=jnp.float32)
                        + bp_ref[...])

    return pl.pallas_call(
        body,
        grid=grid,
        in_specs=[
            pl.BlockSpec((R, D_IN), lambda i: (i, 0)),
            pl.BlockSpec((NC, R, D_IN), lambda i: (0, i, 0)),
            pl.BlockSpec((NC, R, D_IN), lambda i: (0, i, 0)),
            pl.BlockSpec((D_IN, HIDDEN), lambda i: (0, 0)),
            pl.BlockSpec((D_IN, HIDDEN), lambda i: (0, 0)),
            pl.BlockSpec((1, HIDDEN), lambda i: (0, 0)),
            pl.BlockSpec((HIDDEN, HIDDEN), lambda i: (0, 0)),
            pl.BlockSpec((1, HIDDEN), lambda i: (0, 0)),
        ],
        out_specs=pl.BlockSpec((R, HIDDEN), lambda i: (i, 0)),
        out_shape=jax.ShapeDtypeStruct((N_NODES, HIDDEN), jnp.float32),
    )(x, agg2, deg2, W_self, W_neigh, b_conv, W_proj, b_proj)


def kernel(x, edge_index, W_self, W_neigh, b_conv, W_proj, b_proj):
    npad = E_PAD - N_EDGES
    pad_src = jnp.arange(npad, dtype=jnp.int32) % 4096
    pad_dst = N_NODES + (jnp.arange(npad, dtype=jnp.int32) % (N_ACC -
                                                              N_NODES))
    src3 = (jnp.concatenate([edge_index[0], pad_src])
            .reshape(NCHUNK, NW, CHUNK).transpose(1, 0, 2))
    dst3 = (jnp.concatenate([edge_index[1], pad_dst])
            .reshape(NCHUNK, NW, CHUNK).transpose(1, 0, 2))
    zeros_agg = jnp.zeros((N_NODES, D_IN), jnp.float32)
    agg2, deg2 = _sc_agg(x, src3, dst3, zeros_agg)
    return _tc_dense(x, agg2, deg2, W_self, W_neigh,
                     b_conv.reshape(1, HIDDEN), W_proj,
                     b_proj.reshape(1, HIDDEN))
